# baseline (speedup 1.0000x reference)
import functools

import jax
import jax.numpy as jnp
from jax import lax
from jax.experimental import pallas as pl
from jax.experimental.pallas import tpu as pltpu
from jax.experimental.pallas import tpu_sc as plsc

D_MODEL = 256
N_HEADS = 8
N_LEVELS = 4
N_POINTS = 4
D_HEAD = 32
HLP = N_HEADS * N_LEVELS * N_POINTS
LP = N_LEVELS * N_POINTS
NB = 2
LEN_Q = 2048
LEN_IN = 5440
NQ = NB * LEN_Q
NROWS_TABLE = NB * LEN_IN * N_HEADS
NOUT = NQ * N_HEADS
LEVEL_SIZE = (64, 32, 16, 8)
LEVEL_START = (0, 4096, 5120, 5376)

SC_CORES = 2
SC_SUBCORES = 16
NW = SC_CORES * SC_SUBCORES
R_PER_W = NQ // NW
NSLOT = 8
_FIRE_AHEAD = 4

import numpy as _np
_PERM = _np.arange(D_MODEL).reshape(N_HEADS, 2, LP).transpose(0, 2, 1).reshape(-1)

_GSUM = (_np.arange(HLP)[:, None] // LP == _np.arange(HLP)[None, :] // LP
         ).astype(_np.float32)


def _mm_body(x_ref, w_ref, b_ref, o_ref):
    acc = lax.dot_general(x_ref[...], w_ref[...],
                          (((1,), (1,)), ((), ())),
                          preferred_element_type=jnp.float32)
    o_ref[...] = acc + b_ref[0:1, :]


def _mm_bias(x, w, b, bm):
    m, k = x.shape
    n = w.shape[0]
    b8 = jnp.broadcast_to(b.reshape(1, n), (8, n))
    return pl.pallas_call(
        _mm_body,
        grid=(m // bm,),
        in_specs=[
            pl.BlockSpec((bm, k), lambda i: (i, 0)),
            pl.BlockSpec((n, k), lambda i: (0, 0)),
            pl.BlockSpec((8, n), lambda i: (0, 0)),
        ],
        out_specs=pl.BlockSpec((bm, n), lambda i: (i, 0)),
        out_shape=jax.ShapeDtypeStruct((m, n), jnp.float32),
    )(x, w, b8)


def _table_body(x_ref, w_ref, b_ref, o_ref):
    acc = lax.dot_general(x_ref[...], w_ref[...],
                          (((1,), (1,)), ((), ())),
                          preferred_element_type=jnp.float32)
    o_ref[...] = (acc + b_ref[0:1, :]).astype(jnp.bfloat16)


def _value_table(x, w, b, bm):
    m, k = x.shape
    n = w.shape[0]
    b8 = jnp.broadcast_to(b.reshape(1, n), (8, n))
    return pl.pallas_call(
        _table_body,
        grid=(m // bm,),
        in_specs=[
            pl.BlockSpec((bm, k), lambda i: (i, 0)),
            pl.BlockSpec((n, k), lambda i: (0, 0)),
            pl.BlockSpec((8, n), lambda i: (0, 0)),
        ],
        out_specs=pl.BlockSpec((bm, n), lambda i: (i, 0)),
        out_shape=jax.ShapeDtypeStruct((m, n), jnp.bfloat16),
    )(x, w, b8)


_BQ = 2048


def _samp_body(q_ref, m0_ref, m1_ref, rpx_ref, rpy_ref,
               wox_ref, box_ref, woy_ref, boy_ref, wat_ref, bat_ref,
               gsum_ref, pk_ref):
    pid = pl.program_id(0)
    q = q_ref[...]
    dn = (((1,), (1,)), ((), ()))
    offx = lax.dot_general(q, wox_ref[...], dn,
                           preferred_element_type=jnp.float32) + box_ref[0:1, :]
    offy = lax.dot_general(q, woy_ref[...], dn,
                           preferred_element_type=jnp.float32) + boy_ref[0:1, :]
    logit = lax.dot_general(q, wat_ref[...], dn,
                            preferred_element_type=jnp.float32) + bat_ref[0:1, :]
    mask = (m0_ref[...] >= m1_ref[...]).astype(jnp.float32)
    offx = offx * mask
    offy = offy * mask
    logit = logit * mask
    e = jnp.exp(logit)
    s = lax.dot_general(e, gsum_ref[...], (((1,), (1,)), ((), ())),
                        preferred_element_type=jnp.float32)
    aw = e / s

    col = lax.broadcasted_iota(jnp.int32, (_BQ, HLP), 1)
    lvl = (col % LP) // N_POINTS
    head = col // LP
    sz_f = jnp.where(lvl == 0, float(LEVEL_SIZE[0]),
                     jnp.where(lvl == 1, float(LEVEL_SIZE[1]),
                               jnp.where(lvl == 2, float(LEVEL_SIZE[2]),
                                         float(LEVEL_SIZE[3]))))
    start = jnp.where(lvl == 0, LEVEL_START[0],
                      jnp.where(lvl == 1, LEVEL_START[1],
                                jnp.where(lvl == 2, LEVEL_START[2],
                                          LEVEL_START[3])))
    sz_i = sz_f.astype(jnp.int32)

    x = rpx_ref[...] * sz_f + offx - 0.5
    y = rpy_ref[...] * sz_f + offy - 0.5
    x0 = jnp.floor(x)
    y0 = jnp.floor(y)
    wx1 = x - x0
    wx0 = 1.0 - wx1
    wy1 = y - y0
    wy0 = 1.0 - wy1

    row_id = lax.broadcasted_iota(jnp.int32, (_BQ, HLP), 0) + pid * _BQ
    batch_off = jnp.where(row_id >= LEN_Q, LEN_IN, 0)

    sm1 = sz_f - 1.0
    x1 = x0 + 1.0
    y1 = y0 + 1.0
    x0c = jnp.clip(x0, 0.0, sm1)
    x1c = jnp.clip(x1, 0.0, sm1)
    y0c = jnp.clip(y0, 0.0, sm1)
    y1c = jnp.clip(y1, 0.0, sm1)
    wxv0 = wx0 * (x0c == x0).astype(jnp.float32)
    wxv1 = wx1 * (x1c == x1).astype(jnp.float32)
    uy0 = aw * wy0 * (y0c == y0).astype(jnp.float32)
    uy1 = aw * wy1 * (y1c == y1).astype(jnp.float32)
    x0i = x0c.astype(jnp.int32)
    y0i = y0c.astype(jnp.int32)
    base = (batch_off + start + y0i * sz_i + x0i) * N_HEADS + head
    dx = (x1c.astype(jnp.int32) - x0i) * N_HEADS
    dy = (y1c.astype(jnp.int32) - y0i) * (sz_i * N_HEADS)
    bitc = lambda v: lax.bitcast_convert_type(v, jnp.int32)
    pk_ref[0] = jnp.concatenate(
        [base, base + dx, base + dy, base + dy + dx], axis=1)
    pk_ref[1] = jnp.concatenate(
        [bitc(uy0 * wxv0), bitc(uy0 * wxv1),
         bitc(uy1 * wxv0), bitc(uy1 * wxv1)], axis=1)


def _sampling_params(q2, m0, m1, rpx, rpy, wox, box, woy, boy, wat, bat):
    grid = NQ // _BQ
    box = jnp.broadcast_to(box.reshape(1, HLP), (8, HLP))
    boy = jnp.broadcast_to(boy.reshape(1, HLP), (8, HLP))
    bat = jnp.broadcast_to(bat.reshape(1, HLP), (8, HLP))
    full = lambda i: (0, 0)
    rows = lambda i: (i, 0)
    return pl.pallas_call(
        _samp_body,
        grid=(grid,),
        in_specs=[
            pl.BlockSpec((_BQ, D_MODEL), rows),
            pl.BlockSpec((_BQ, HLP), rows),
            pl.BlockSpec((_BQ, HLP), rows),
            pl.BlockSpec((_BQ, HLP), rows),
            pl.BlockSpec((_BQ, HLP), rows),
            pl.BlockSpec((HLP, D_MODEL), full),
            pl.BlockSpec((8, HLP), full),
            pl.BlockSpec((HLP, D_MODEL), full),
            pl.BlockSpec((8, HLP), full),
            pl.BlockSpec((HLP, D_MODEL), full),
            pl.BlockSpec((8, HLP), full),
            pl.BlockSpec((HLP, HLP), full),
        ],
        out_specs=pl.BlockSpec((2, _BQ, 4 * HLP), lambda i: (0, i, 0)),
        out_shape=jax.ShapeDtypeStruct((2, NQ, 4 * HLP), jnp.int32),
    )(q2, m0, m1, rpx, rpy, wox, box, woy, boy, wat, bat,
      jnp.asarray(_GSUM))


_TQ = 1
_NCH = R_PER_W // _TQ
_CW = _TQ * 4 * HLP


def _sc_body(table_h, pk_h, out_h, pkv, bufv, outv, *sems):
    wid = lax.axis_index("s") * SC_CORES + lax.axis_index("c")
    base_r = wid * R_PER_W
    semg = sems[:NSLOT]
    semi = sems[NSLOT:]

    def issue_pk(chunk, s):
        pltpu.async_copy(pk_h.at[:, base_r + chunk], pkv.at[s], semi[s])

    def fire(chunk, s):
        pltpu.make_async_copy(pk_h.at[:, base_r + chunk], pkv.at[s],
                              semi[s]).wait()
        pltpu.async_copy(table_h.at[pkv.at[s, 0]], bufv.at[s], semg[s])

    def drain(s):
        pltpu.make_async_copy(table_h.at[pkv.at[s, 0]], bufv.at[s],
                              semg[s]).wait()

    def compute(chunk, s):
        def h_body(h, _):
            hb = h * LP
            for t in range(_TQ):
                tb = t * 4 * HLP
                acc0 = jnp.zeros((16,), jnp.float32)
                acc1 = jnp.zeros((16,), jnp.float32)
                for c in range(4):
                    wvec = plsc.bitcast(
                        pkv[s, 1, pl.ds(tb + c * HLP + hb, LP)], jnp.float32)
                    for p in range(LP):
                        wbc = jnp.take_along_axis(
                            wvec, jnp.full((16,), p, jnp.int32), axis=0)
                        v32 = bufv[s, tb + c * HLP + hb + p, :]
                        lo, hi = plsc.unpack(
                            v32, format=plsc.PackFormat.INTERLEAVED)
                        acc0 = acc0 + wbc * lo
                        acc1 = acc1 + wbc * hi
                orow = (chunk * _TQ + t) * N_HEADS + h
                outv[orow, pl.ds(0, 16)] = acc0
                outv[orow, pl.ds(16, 16)] = acc1
            return 0
        lax.fori_loop(0, N_HEADS, h_body, 0)

    for k in range(NSLOT - 1):
        issue_pk(k, k)
    for k in range(_FIRE_AHEAD):
        fire(k, k)

    def loop(ii, _):
        for b in range(NSLOT):
            chunk = ii * NSLOT + b
            drain(b)
            compute(chunk, b)

            @pl.when(chunk + (NSLOT - 1) < _NCH)
            def _():
                issue_pk(chunk + (NSLOT - 1), (b + NSLOT - 1) % NSLOT)

            @pl.when(chunk + _FIRE_AHEAD < _NCH)
            def _():
                fire(chunk + _FIRE_AHEAD, (b + _FIRE_AHEAD) % NSLOT)
        return 0

    lax.fori_loop(0, _NCH // NSLOT, loop, 0)
    pltpu.sync_copy(outv, out_h.at[pl.ds(wid * R_PER_W * N_HEADS,
                                         R_PER_W * N_HEADS)])


def _sc_gather_sum(table, pk):
    mesh = plsc.VectorSubcoreMesh(core_axis_name="c", subcore_axis_name="s",
                                  num_cores=SC_CORES, num_subcores=SC_SUBCORES)
    f = functools.partial(
        pl.kernel,
        out_type=jax.ShapeDtypeStruct((NOUT, D_HEAD), jnp.float32),
        mesh=mesh,
        scratch_types=[
            pltpu.VMEM((NSLOT, 2, _CW), jnp.int32),
            pltpu.VMEM((NSLOT, _CW, D_HEAD), jnp.bfloat16),
            pltpu.VMEM((R_PER_W * N_HEADS, D_HEAD), jnp.float32),
        ] + [pltpu.SemaphoreType.DMA] * (2 * NSLOT),
        compiler_params=pltpu.CompilerParams(use_tc_tiling_on_sc=False,
                                             needs_layout_passes=False),
    )(_sc_body)
    return f(table, pk)


def kernel(mask_num, dis, query, reference_points, input_flatten,
           input_spatial_shapes, input_level_start_index,
           W_off, b_off, W_attn, b_attn, W_val, b_val, W_out, b_out):
    q2 = query.reshape(NQ, D_MODEL)
    x2 = input_flatten.reshape(NB * LEN_IN, D_MODEL)
    m0 = mask_num[:, 0].reshape(NQ, HLP)
    m1 = mask_num[:, 1].reshape(NQ, HLP)
    rp = reference_points.reshape(NQ, N_LEVELS, 2)
    rpx = jnp.tile(jnp.repeat(rp[:, :, 0], N_POINTS, axis=1), (1, N_HEADS))
    rpy = jnp.tile(jnp.repeat(rp[:, :, 1], N_POINTS, axis=1), (1, N_HEADS))
    d0 = dis[0]
    wox = W_off[0::2] * d0
    woy = W_off[1::2] * d0
    box = b_off[0::2] * d0
    boy = b_off[1::2] * d0

    table = _value_table(x2, W_val[_PERM], b_val[_PERM],
                         bm=5440).reshape(NROWS_TABLE, D_HEAD)
    pk = _sampling_params(q2, m0, m1, rpx, rpy,
                          wox, box, woy, boy, W_attn, b_attn)
    core = _sc_gather_sum(table, pk)
    out = _mm_bias(core.reshape(NQ, D_MODEL), W_out, b_out, bm=4096)
    return out.reshape(NB, LEN_Q, D_MODEL)


# speedup vs baseline: 1.0047x; 1.0047x over previous
import functools

import jax
import jax.numpy as jnp
from jax import lax
from jax.experimental import pallas as pl
from jax.experimental.pallas import tpu as pltpu
from jax.experimental.pallas import tpu_sc as plsc

D_MODEL = 256
N_HEADS = 8
N_LEVELS = 4
N_POINTS = 4
D_HEAD = 32
HLP = N_HEADS * N_LEVELS * N_POINTS
LP = N_LEVELS * N_POINTS
NB = 2
LEN_Q = 2048
LEN_IN = 5440
NQ = NB * LEN_Q
NROWS_TABLE = NB * LEN_IN * N_HEADS
NOUT = NQ * N_HEADS
LEVEL_SIZE = (64, 32, 16, 8)
LEVEL_START = (0, 4096, 5120, 5376)

SC_CORES = 2
SC_SUBCORES = 16
NW = SC_CORES * SC_SUBCORES
R_PER_W = NQ // NW
NSLOT = 8
_FIRE_AHEAD = 4

import numpy as _np
_PERM = _np.arange(D_MODEL).reshape(N_HEADS, 2, LP).transpose(0, 2, 1).reshape(-1)

_GSUM = (_np.arange(HLP)[:, None] // LP == _np.arange(HLP)[None, :] // LP
         ).astype(_np.float32)


def _mm_body(x_ref, w_ref, b_ref, o_ref):
    acc = lax.dot_general(x_ref[...], w_ref[...],
                          (((1,), (1,)), ((), ())),
                          preferred_element_type=jnp.float32)
    o_ref[...] = acc + b_ref[0:1, :]


def _mm_bias(x, w, b, bm):
    m, k = x.shape
    n = w.shape[0]
    b8 = jnp.broadcast_to(b.reshape(1, n), (8, n))
    return pl.pallas_call(
        _mm_body,
        grid=(m // bm,),
        in_specs=[
            pl.BlockSpec((bm, k), lambda i: (i, 0)),
            pl.BlockSpec((n, k), lambda i: (0, 0)),
            pl.BlockSpec((8, n), lambda i: (0, 0)),
        ],
        out_specs=pl.BlockSpec((bm, n), lambda i: (i, 0)),
        out_shape=jax.ShapeDtypeStruct((m, n), jnp.float32),
    )(x, w, b8)


def _table_body(x_ref, w_ref, b_ref, o_ref):
    acc = lax.dot_general(x_ref[...], w_ref[...],
                          (((1,), (1,)), ((), ())),
                          preferred_element_type=jnp.float32)
    o_ref[...] = (acc + b_ref[0:1, :]).astype(jnp.bfloat16)


def _value_table(x, w, b, bm):
    m, k = x.shape
    n = w.shape[0]
    b8 = jnp.broadcast_to(b.reshape(1, n), (8, n))
    return pl.pallas_call(
        _table_body,
        grid=(m // bm,),
        in_specs=[
            pl.BlockSpec((bm, k), lambda i: (i, 0)),
            pl.BlockSpec((n, k), lambda i: (0, 0)),
            pl.BlockSpec((8, n), lambda i: (0, 0)),
        ],
        out_specs=pl.BlockSpec((bm, n), lambda i: (i, 0)),
        out_shape=jax.ShapeDtypeStruct((m, n), jnp.bfloat16),
    )(x, w, b8)


_BQ = 1024


def _samp_body(q_ref, m0_ref, m1_ref, rpx_ref, rpy_ref,
               wox_ref, box_ref, woy_ref, boy_ref, wat_ref, bat_ref,
               gsum_ref, pk_ref):
    pid = pl.program_id(0)
    q = q_ref[...]
    dn = (((1,), (1,)), ((), ()))
    offx = lax.dot_general(q, wox_ref[...], dn,
                           preferred_element_type=jnp.float32) + box_ref[0:1, :]
    offy = lax.dot_general(q, woy_ref[...], dn,
                           preferred_element_type=jnp.float32) + boy_ref[0:1, :]
    logit = lax.dot_general(q, wat_ref[...], dn,
                            preferred_element_type=jnp.float32) + bat_ref[0:1, :]
    mask = (m0_ref[...] >= m1_ref[...]).astype(jnp.float32)
    offx = offx * mask
    offy = offy * mask
    logit = logit * mask
    e = jnp.exp(logit)
    s = lax.dot_general(e, gsum_ref[...], (((1,), (1,)), ((), ())),
                        preferred_element_type=jnp.float32)
    aw = e / s

    col = lax.broadcasted_iota(jnp.int32, (_BQ, HLP), 1)
    lvl = (col % LP) // N_POINTS
    head = col // LP
    sz_f = jnp.where(lvl == 0, float(LEVEL_SIZE[0]),
                     jnp.where(lvl == 1, float(LEVEL_SIZE[1]),
                               jnp.where(lvl == 2, float(LEVEL_SIZE[2]),
                                         float(LEVEL_SIZE[3]))))
    start = jnp.where(lvl == 0, LEVEL_START[0],
                      jnp.where(lvl == 1, LEVEL_START[1],
                                jnp.where(lvl == 2, LEVEL_START[2],
                                          LEVEL_START[3])))
    sz_i = sz_f.astype(jnp.int32)

    x = rpx_ref[...] * sz_f + offx - 0.5
    y = rpy_ref[...] * sz_f + offy - 0.5
    x0 = jnp.floor(x)
    y0 = jnp.floor(y)
    wx1 = x - x0
    wx0 = 1.0 - wx1
    wy1 = y - y0
    wy0 = 1.0 - wy1

    row_id = lax.broadcasted_iota(jnp.int32, (_BQ, HLP), 0) + pid * _BQ
    batch_off = jnp.where(row_id >= LEN_Q, LEN_IN, 0)

    sm1 = sz_f - 1.0
    x1 = x0 + 1.0
    y1 = y0 + 1.0
    x0c = jnp.clip(x0, 0.0, sm1)
    x1c = jnp.clip(x1, 0.0, sm1)
    y0c = jnp.clip(y0, 0.0, sm1)
    y1c = jnp.clip(y1, 0.0, sm1)
    wxv0 = wx0 * (x0c == x0).astype(jnp.float32)
    wxv1 = wx1 * (x1c == x1).astype(jnp.float32)
    uy0 = aw * wy0 * (y0c == y0).astype(jnp.float32)
    uy1 = aw * wy1 * (y1c == y1).astype(jnp.float32)
    x0i = x0c.astype(jnp.int32)
    y0i = y0c.astype(jnp.int32)
    base = (batch_off + start + y0i * sz_i + x0i) * N_HEADS + head
    dx = (x1c.astype(jnp.int32) - x0i) * N_HEADS
    dy = (y1c.astype(jnp.int32) - y0i) * (sz_i * N_HEADS)
    bitc = lambda v: lax.bitcast_convert_type(v, jnp.int32)
    pk_ref[0] = jnp.concatenate(
        [base, base + dx, base + dy, base + dy + dx], axis=1)
    pk_ref[1] = jnp.concatenate(
        [bitc(uy0 * wxv0), bitc(uy0 * wxv1),
         bitc(uy1 * wxv0), bitc(uy1 * wxv1)], axis=1)


def _sampling_params(q2, m0, m1, rpx, rpy, wox, box, woy, boy, wat, bat):
    grid = NQ // _BQ
    box = jnp.broadcast_to(box.reshape(1, HLP), (8, HLP))
    boy = jnp.broadcast_to(boy.reshape(1, HLP), (8, HLP))
    bat = jnp.broadcast_to(bat.reshape(1, HLP), (8, HLP))
    full = lambda i: (0, 0)
    rows = lambda i: (i, 0)
    return pl.pallas_call(
        _samp_body,
        grid=(grid,),
        in_specs=[
            pl.BlockSpec((_BQ, D_MODEL), rows),
            pl.BlockSpec((_BQ, HLP), rows),
            pl.BlockSpec((_BQ, HLP), rows),
            pl.BlockSpec((_BQ, HLP), rows),
            pl.BlockSpec((_BQ, HLP), rows),
            pl.BlockSpec((HLP, D_MODEL), full),
            pl.BlockSpec((8, HLP), full),
            pl.BlockSpec((HLP, D_MODEL), full),
            pl.BlockSpec((8, HLP), full),
            pl.BlockSpec((HLP, D_MODEL), full),
            pl.BlockSpec((8, HLP), full),
            pl.BlockSpec((HLP, HLP), full),
        ],
        out_specs=pl.BlockSpec((2, _BQ, 4 * HLP), lambda i: (0, i, 0)),
        out_shape=jax.ShapeDtypeStruct((2, NQ, 4 * HLP), jnp.int32),
    )(q2, m0, m1, rpx, rpy, wox, box, woy, boy, wat, bat,
      jnp.asarray(_GSUM))


_TQ = 1
_NCH = R_PER_W // _TQ
_CW = _TQ * 4 * HLP


def _sc_body(table_h, pk_h, out_h, pkv, bufv, outv, *sems):
    wid = lax.axis_index("s") * SC_CORES + lax.axis_index("c")
    base_r = wid * R_PER_W
    semg = sems[:NSLOT]
    semi = sems[NSLOT:]

    def issue_pk(chunk, s):
        pltpu.async_copy(pk_h.at[:, base_r + chunk], pkv.at[s], semi[s])

    def fire(chunk, s):
        pltpu.make_async_copy(pk_h.at[:, base_r + chunk], pkv.at[s],
                              semi[s]).wait()
        pltpu.async_copy(table_h.at[pkv.at[s, 0]], bufv.at[s], semg[s])

    def drain(s):
        pltpu.make_async_copy(table_h.at[pkv.at[s, 0]], bufv.at[s],
                              semg[s]).wait()

    def compute(chunk, s):
        def h_body(h, _):
            hb = h * LP
            for t in range(_TQ):
                tb = t * 4 * HLP
                acc0 = jnp.zeros((16,), jnp.float32)
                acc1 = jnp.zeros((16,), jnp.float32)
                for c in range(4):
                    wvec = plsc.bitcast(
                        pkv[s, 1, pl.ds(tb + c * HLP + hb, LP)], jnp.float32)
                    for p in range(LP):
                        wbc = jnp.take_along_axis(
                            wvec, jnp.full((16,), p, jnp.int32), axis=0)
                        v32 = bufv[s, tb + c * HLP + hb + p, :]
                        lo, hi = plsc.unpack(
                            v32, format=plsc.PackFormat.INTERLEAVED)
                        acc0 = acc0 + wbc * lo
                        acc1 = acc1 + wbc * hi
                orow = (chunk * _TQ + t) * N_HEADS + h
                outv[orow, pl.ds(0, 16)] = acc0
                outv[orow, pl.ds(16, 16)] = acc1
            return 0
        lax.fori_loop(0, N_HEADS, h_body, 0)

    for k in range(NSLOT - 1):
        issue_pk(k, k)
    for k in range(_FIRE_AHEAD):
        fire(k, k)

    def loop(ii, _):
        for b in range(NSLOT):
            chunk = ii * NSLOT + b
            drain(b)
            compute(chunk, b)

            @pl.when(chunk + (NSLOT - 1) < _NCH)
            def _():
                issue_pk(chunk + (NSLOT - 1), (b + NSLOT - 1) % NSLOT)

            @pl.when(chunk + _FIRE_AHEAD < _NCH)
            def _():
                fire(chunk + _FIRE_AHEAD, (b + _FIRE_AHEAD) % NSLOT)
        return 0

    lax.fori_loop(0, _NCH // NSLOT, loop, 0)
    pltpu.sync_copy(outv, out_h.at[pl.ds(wid * R_PER_W * N_HEADS,
                                         R_PER_W * N_HEADS)])


def _sc_gather_sum(table, pk):
    mesh = plsc.VectorSubcoreMesh(core_axis_name="c", subcore_axis_name="s",
                                  num_cores=SC_CORES, num_subcores=SC_SUBCORES)
    f = functools.partial(
        pl.kernel,
        out_type=jax.ShapeDtypeStruct((NOUT, D_HEAD), jnp.float32),
        mesh=mesh,
        scratch_types=[
            pltpu.VMEM((NSLOT, 2, _CW), jnp.int32),
            pltpu.VMEM((NSLOT, _CW, D_HEAD), jnp.bfloat16),
            pltpu.VMEM((R_PER_W * N_HEADS, D_HEAD), jnp.float32),
        ] + [pltpu.SemaphoreType.DMA] * (2 * NSLOT),
        compiler_params=pltpu.CompilerParams(use_tc_tiling_on_sc=False,
                                             needs_layout_passes=False),
    )(_sc_body)
    return f(table, pk)


def kernel(mask_num, dis, query, reference_points, input_flatten,
           input_spatial_shapes, input_level_start_index,
           W_off, b_off, W_attn, b_attn, W_val, b_val, W_out, b_out):
    q2 = query.reshape(NQ, D_MODEL)
    x2 = input_flatten.reshape(NB * LEN_IN, D_MODEL)
    m0 = mask_num[:, 0].reshape(NQ, HLP)
    m1 = mask_num[:, 1].reshape(NQ, HLP)
    rp = reference_points.reshape(NQ, N_LEVELS, 2)
    rpx = jnp.tile(jnp.repeat(rp[:, :, 0], N_POINTS, axis=1), (1, N_HEADS))
    rpy = jnp.tile(jnp.repeat(rp[:, :, 1], N_POINTS, axis=1), (1, N_HEADS))
    d0 = dis[0]
    wox = W_off[0::2] * d0
    woy = W_off[1::2] * d0
    box = b_off[0::2] * d0
    boy = b_off[1::2] * d0

    table = _value_table(x2, W_val[_PERM], b_val[_PERM],
                         bm=2176).reshape(NROWS_TABLE, D_HEAD)
    pk = _sampling_params(q2, m0, m1, rpx, rpy,
                          wox, box, woy, boy, W_attn, b_attn)
    core = _sc_gather_sum(table, pk)
    out = _mm_bias(core.reshape(NQ, D_MODEL), W_out, b_out, bm=2048)
    return out.reshape(NB, LEN_Q, D_MODEL)
